# baseline (speedup 1.0000x reference)
import functools

import jax
import jax.numpy as jnp
from jax import lax
from jax.experimental import pallas as pl
from jax.experimental.pallas import tpu as pltpu
from jax.experimental.pallas import tpu_sc as plsc

N = 320000
D = 128
SEG = 10000
SEGP = 10240
PADSEG = SEGP - 1
NC = 2
NS = 16
NW = NC * NS
ROWS_W = N // NW
C = 80
K = ROWS_W // C
G = C // 16
SEG_T = SEGP // NS
NV = D // 16
U = 4


def _sc_body(emb_hbm, idx_hbm, out_hbm, idx_v, buf0, buf1, cbuf, cidx, idx_w,
             acc, sem0, sem1):
    c = lax.axis_index("c")
    s = lax.axis_index("s")
    wid = c * NS + s
    row_base = wid * ROWS_W

    zeros = jnp.zeros((16,), jnp.float32)
    pad = jnp.full((16,), PADSEG, jnp.int32)

    def _zrow(i, _):
        for j in range(NV):
            cbuf[i, pl.ds(j * 16, 16)] = zeros
        return 0

    idx_w[pl.ds(0, 16)] = jnp.full((16,), -1, jnp.int32)
    lax.fori_loop(0, C, _zrow, 0)
    for r in range(SEG_T // C):
        pltpu.sync_copy(cbuf, acc.at[pl.ds(s * SEG_T + r * C, C)])

    pltpu.sync_copy(idx_hbm.at[wid], idx_v)

    plsc.subcore_barrier()

    def _gather(g, buf, sem):
        pltpu.async_copy(emb_hbm.at[pl.ds(row_base + g * C, C)], buf, sem)

    def _gwait(buf, sem):
        pltpu.make_async_copy(emb_hbm.at[pl.ds(row_base, C)], buf, sem).wait()

    lanes = jnp.arange(16, dtype=jnp.int32)

    def _consume(g, buf):
        for t in range(G):
            idx_w[pl.ds(16 + 16 * t, 16)] = idx_v[g, pl.ds(16 * t, 16)]

        def _kgrp(k, carry):
            posc, prev, cidvec = carry[0], carry[1], carry[2]
            a = list(carry[3:])
            base = 16 * k
            for u in range(16):
                iv0 = idx_w[pl.ds(base + 16 + u, 16)][0]
                same = iv0 == prev
                posc = posc + jnp.where(same, 0, 1)
                newgrp = jnp.logical_and((posc & 15) == 0,
                                         jnp.logical_not(same))
                cidvec = jnp.where(newgrp, pad, cidvec)
                cidvec = jnp.where(lanes == (posc & 15), iv0, cidvec)
                cidx[posc >> 4, :] = cidvec
                for j in range(NV):
                    r = buf[base + u, pl.ds(j * 16, 16)]
                    a[j] = jnp.where(same, a[j], zeros) + r
                    cbuf[posc, pl.ds(j * 16, 16)] = a[j]
                prev = iv0
            return (posc, prev, cidvec, *a)

        init = (jnp.int32(-1), jnp.int32(-1), pad) + tuple([zeros] * NV)
        pos = lax.fori_loop(0, G, _kgrp, init)[0]

        def _flush(j, _):
            pltpu.sync_copy(
                cbuf.at[pl.ds(j * 16, 16)], acc.at[cidx.at[j]], add=True
            )
            return 0

        lax.fori_loop(0, (pos >> 4) + 1, _flush, 0)

    _gather(0, buf0, sem0)

    def _pair(i, _):
        g = 2 * i
        _gather(g + 1, buf1, sem1)
        _gwait(buf0, sem0)
        _consume(g, buf0)
        _gather(g + 2, buf0, sem0)
        _gwait(buf1, sem1)
        _consume(g + 1, buf1)
        return 0

    lax.fori_loop(0, (K - 1) // 2, _pair, 0)
    _gwait(buf0, sem0)
    _consume(K - 1, buf0)

    plsc.subcore_barrier()

    pltpu.sync_copy(
        acc.at[pl.ds(s * SEG_T, SEG_T)],
        out_hbm.at[c, pl.ds(s * SEG_T, SEG_T)],
    )


@functools.partial(
    pl.kernel,
    mesh=plsc.VectorSubcoreMesh(core_axis_name="c", subcore_axis_name="s"),
    out_type=jax.ShapeDtypeStruct((NC, SEGP, D), jnp.float32),
    scratch_types=[
        pltpu.VMEM((K, C), jnp.int32),
        pltpu.VMEM((C, D), jnp.float32),
        pltpu.VMEM((C, D), jnp.float32),
        pltpu.VMEM((C, D), jnp.float32),
        pltpu.VMEM((G, 16), jnp.int32),
        pltpu.VMEM((C + 16,), jnp.int32),
        pltpu.VMEM_SHARED((SEGP, D), jnp.float32),
        pltpu.SemaphoreType.DMA,
        pltpu.SemaphoreType.DMA,
    ],
)
def _sc_segment_sum(emb_hbm, idx_hbm, out_hbm, idx_v, buf0, buf1, cbuf, cidx,
                    idx_w, acc, sem0, sem1):
    _sc_body(emb_hbm, idx_hbm, out_hbm, idx_v, buf0, buf1, cbuf, cidx, idx_w,
             acc, sem0, sem1)


def _combine_body(a_ref, b_ref, o_ref):
    o_ref[...] = a_ref[...] + b_ref[...]


def kernel(embedding, fragment_regionxcell_ix, cell_n, region_n):
    del cell_n, region_n
    idx3 = fragment_regionxcell_ix.reshape(NW, K, C)
    partials = _sc_segment_sum(embedding, idx3)
    out = pl.pallas_call(
        _combine_body,
        grid=(10,),
        in_specs=[
            pl.BlockSpec((SEG // 10, D), lambda i: (i, 0)),
            pl.BlockSpec((SEG // 10, D), lambda i: (i, 0)),
        ],
        out_specs=pl.BlockSpec((SEG // 10, D), lambda i: (i, 0)),
        out_shape=jax.ShapeDtypeStruct((SEG, D), jnp.float32),
    )(partials[0], partials[1])
    return out.reshape(10, 1000, D)


# speedup vs baseline: 3.3971x; 3.3971x over previous
import functools

import jax
import jax.numpy as jnp
from jax import lax
from jax.experimental import pallas as pl
from jax.experimental.pallas import tpu as pltpu
from jax.experimental.pallas import tpu_sc as plsc

N = 320000
D = 128
SEG = 10000
SEGP = 10240
NC = 2
NS = 16
NW = NC * NS
ROWS_W = N // NW
C = 80
K = ROWS_W // C
SEG_T = SEGP // NS


def _sc_body(emb_hbm, idx_hbm, out_hbm, idx_v, buf0, buf1, acc, sem0, sem1,
             ssem0, ssem1):
    c = lax.axis_index("c")
    s = lax.axis_index("s")
    wid = c * NS + s
    row_base = wid * ROWS_W

    zeros = jnp.zeros((16,), jnp.float32)

    def _zrow(i, _):
        for j in range(D // 16):
            buf0[i, pl.ds(j * 16, 16)] = zeros
        return 0

    lax.fori_loop(0, C, _zrow, 0)
    for r in range(SEG_T // C):
        pltpu.sync_copy(buf0, acc.at[pl.ds(s * SEG_T + r * C, C)])

    pltpu.sync_copy(idx_hbm.at[wid], idx_v)

    plsc.subcore_barrier()

    def _gather(g, buf, sem):
        pltpu.async_copy(emb_hbm.at[pl.ds(row_base + g * C, C)], buf, sem)

    def _gwait(buf, sem):
        pltpu.make_async_copy(emb_hbm.at[pl.ds(row_base, C)], buf, sem).wait()

    def _scat(g, buf, sem):
        pltpu.async_copy(buf, acc.at[idx_v.at[g]], sem, add=True)

    def _swait(buf, sem):
        pltpu.make_async_copy(buf, acc.at[idx_v.at[0]], sem).wait()

    _gather(0, buf0, sem0)
    _gwait(buf0, sem0)
    _scat(0, buf0, ssem0)
    _gather(1, buf1, sem1)

    def _pair(i, _):
        g = 2 * i
        _gwait(buf1, sem1)
        _scat(g + 1, buf1, ssem1)
        _swait(buf0, ssem0)
        _gather(g + 2, buf0, sem0)
        _gwait(buf0, sem0)
        _scat(g + 2, buf0, ssem0)
        _swait(buf1, ssem1)
        _gather(g + 3, buf1, sem1)
        return 0

    lax.fori_loop(0, (K - 3) // 2, _pair, 0)
    _gwait(buf1, sem1)
    _scat(K - 2, buf1, ssem1)
    _swait(buf0, ssem0)
    _gather(K - 1, buf0, sem0)
    _gwait(buf0, sem0)
    _scat(K - 1, buf0, ssem0)
    _swait(buf0, ssem0)
    _swait(buf1, ssem1)

    plsc.subcore_barrier()

    pltpu.sync_copy(
        acc.at[pl.ds(s * SEG_T, SEG_T)],
        out_hbm.at[c, pl.ds(s * SEG_T, SEG_T)],
    )


@functools.partial(
    pl.kernel,
    mesh=plsc.VectorSubcoreMesh(core_axis_name="c", subcore_axis_name="s"),
    out_type=jax.ShapeDtypeStruct((NC, SEGP, D), jnp.float32),
    scratch_types=[
        pltpu.VMEM((K, C), jnp.int32),
        pltpu.VMEM((C, D), jnp.float32),
        pltpu.VMEM((C, D), jnp.float32),
        pltpu.VMEM_SHARED((SEGP, D), jnp.float32),
        pltpu.SemaphoreType.DMA,
        pltpu.SemaphoreType.DMA,
        pltpu.SemaphoreType.DMA,
        pltpu.SemaphoreType.DMA,
    ],
)
def _sc_segment_sum(emb_hbm, idx_hbm, out_hbm, idx_v, buf0, buf1, acc, sem0,
                    sem1, ssem0, ssem1):
    _sc_body(emb_hbm, idx_hbm, out_hbm, idx_v, buf0, buf1, acc, sem0, sem1,
             ssem0, ssem1)


def _combine_body(a_ref, b_ref, o_ref):
    o_ref[...] = a_ref[...] + b_ref[...]


def kernel(embedding, fragment_regionxcell_ix, cell_n, region_n):
    del cell_n, region_n
    idx3 = fragment_regionxcell_ix.reshape(NW, K, C)
    partials = _sc_segment_sum(embedding, idx3)
    out = pl.pallas_call(
        _combine_body,
        grid=(10,),
        in_specs=[
            pl.BlockSpec((SEG // 10, D), lambda i: (i, 0)),
            pl.BlockSpec((SEG // 10, D), lambda i: (i, 0)),
        ],
        out_specs=pl.BlockSpec((SEG // 10, D), lambda i: (i, 0)),
        out_shape=jax.ShapeDtypeStruct((SEG, D), jnp.float32),
    )(partials[0], partials[1])
    return out.reshape(10, 1000, D)


# speedup vs baseline: 4.1564x; 1.2235x over previous
import functools

import jax
import jax.numpy as jnp
from jax import lax
from jax.experimental import pallas as pl
from jax.experimental.pallas import tpu as pltpu
from jax.experimental.pallas import tpu_sc as plsc

N = 320000
D = 128
SEG = 10000
SEGP = 10240
NC = 2
NS = 16
NW = NC * NS
ROWS_W = N // NW
C = 80
K = ROWS_W // C
SEG_T = SEGP // NS


def _sc_body(emb_hbm, idx_hbm, out_hbm, idx_v, buf0, buf1, acc, sem0, sem1):
    c = lax.axis_index("c")
    s = lax.axis_index("s")
    wid = c * NS + s
    row_base = wid * ROWS_W

    zeros = jnp.zeros((16,), jnp.float32)

    def _zrow(i, _):
        for j in range(D // 16):
            buf0[i, pl.ds(j * 16, 16)] = zeros
        return 0

    lax.fori_loop(0, C, _zrow, 0)
    for r in range(SEG_T // C):
        pltpu.sync_copy(buf0, acc.at[pl.ds(s * SEG_T + r * C, C)])

    pltpu.sync_copy(idx_hbm.at[wid], idx_v)

    plsc.subcore_barrier()

    def _gather(g, buf, sem):
        pltpu.async_copy(emb_hbm.at[pl.ds(row_base + g * C, C)], buf, sem)

    def _gwait(buf, sem):
        pltpu.make_async_copy(emb_hbm.at[pl.ds(row_base, C)], buf, sem).wait()

    def _scat(g, buf):
        pltpu.sync_copy(buf, acc.at[idx_v.at[g]], add=True)

    _gather(0, buf0, sem0)

    def _pair(i, _):
        g = 2 * i
        _gather(g + 1, buf1, sem1)
        _gwait(buf0, sem0)
        _scat(g, buf0)
        _gather(g + 2, buf0, sem0)
        _gwait(buf1, sem1)
        _scat(g + 1, buf1)
        return 0

    lax.fori_loop(0, (K - 1) // 2, _pair, 0)
    _gwait(buf0, sem0)
    _scat(K - 1, buf0)

    plsc.subcore_barrier()

    pltpu.sync_copy(
        acc.at[pl.ds(s * SEG_T, SEG_T)],
        out_hbm.at[c, pl.ds(s * SEG_T, SEG_T)],
    )


@functools.partial(
    pl.kernel,
    mesh=plsc.VectorSubcoreMesh(core_axis_name="c", subcore_axis_name="s"),
    out_type=jax.ShapeDtypeStruct((NC, SEGP, D), jnp.float32),
    scratch_types=[
        pltpu.VMEM((K, C), jnp.int32),
        pltpu.VMEM((C, D), jnp.float32),
        pltpu.VMEM((C, D), jnp.float32),
        pltpu.VMEM_SHARED((SEGP, D), jnp.float32),
        pltpu.SemaphoreType.DMA,
        pltpu.SemaphoreType.DMA,
    ],
)
def _sc_segment_sum(emb_hbm, idx_hbm, out_hbm, idx_v, buf0, buf1, acc, sem0, sem1):
    _sc_body(emb_hbm, idx_hbm, out_hbm, idx_v, buf0, buf1, acc, sem0, sem1)


def _combine_body(a_ref, b_ref, o_ref):
    o_ref[...] = a_ref[...] + b_ref[...]


def kernel(embedding, fragment_regionxcell_ix, cell_n, region_n):
    del cell_n, region_n
    idx3 = fragment_regionxcell_ix.reshape(NW, K, C)
    partials = _sc_segment_sum(embedding, idx3)
    out = pl.pallas_call(
        _combine_body,
        grid=(10,),
        in_specs=[
            pl.BlockSpec((SEG // 10, D), lambda i: (i, 0)),
            pl.BlockSpec((SEG // 10, D), lambda i: (i, 0)),
        ],
        out_specs=pl.BlockSpec((SEG // 10, D), lambda i: (i, 0)),
        out_shape=jax.ShapeDtypeStruct((SEG, D), jnp.float32),
    )(partials[0], partials[1])
    return out.reshape(10, 1000, D)


# speedup vs baseline: 4.3509x; 1.0468x over previous
import functools

import jax
import jax.numpy as jnp
from jax import lax
from jax.experimental import pallas as pl
from jax.experimental.pallas import tpu as pltpu
from jax.experimental.pallas import tpu_sc as plsc

N = 320000
D = 128
SEG = 10000
SEGP = 10240
NC = 2
NS = 16
NW = NC * NS
ROWS_W = N // NW
C = 80
K = ROWS_W // C
SEG_T = SEGP // NS


def _sc_body(emb_hbm, idx_hbm, out_hbm, idx_v, buf0, buf1, acc, sem0, sem1):
    c = lax.axis_index("c")
    s = lax.axis_index("s")
    wid = c * NS + s
    row_base = wid * ROWS_W

    zeros = jnp.zeros((16,), jnp.float32)

    def _zrow(i, _):
        for j in range(D // 16):
            buf0[i, pl.ds(j * 16, 16)] = zeros
        return 0

    lax.fori_loop(0, C, _zrow, 0)
    for r in range(SEG_T // C):
        pltpu.sync_copy(buf0, acc.at[pl.ds(s * SEG_T + r * C, C)])

    pltpu.sync_copy(idx_hbm.at[wid], idx_v)

    plsc.subcore_barrier()

    def _gather(g, buf, sem):
        pltpu.async_copy(emb_hbm.at[pl.ds(row_base + g * C, C)], buf, sem)

    def _gwait(buf, sem):
        pltpu.make_async_copy(emb_hbm.at[pl.ds(row_base, C)], buf, sem).wait()

    def _scat(g, buf):
        pltpu.sync_copy(buf, acc.at[idx_v.at[g]], add=True)

    _gather(0, buf0, sem0)

    def _pair(i, _):
        g = 2 * i
        _gather(g + 1, buf1, sem1)
        _gwait(buf0, sem0)
        _scat(g, buf0)
        _gather(g + 2, buf0, sem0)
        _gwait(buf1, sem1)
        _scat(g + 1, buf1)
        return 0

    lax.fori_loop(0, (K - 1) // 2, _pair, 0)
    _gwait(buf0, sem0)
    _scat(K - 1, buf0)

    plsc.subcore_barrier()

    pltpu.sync_copy(
        acc.at[pl.ds(s * SEG_T, SEG_T)],
        out_hbm.at[c, pl.ds(s * SEG_T, SEG_T)],
    )


@functools.partial(
    pl.kernel,
    mesh=plsc.VectorSubcoreMesh(core_axis_name="c", subcore_axis_name="s"),
    out_type=jax.ShapeDtypeStruct((NC, SEGP, D), jnp.float32),
    scratch_types=[
        pltpu.VMEM((K, C), jnp.int32),
        pltpu.VMEM((C, D), jnp.float32),
        pltpu.VMEM((C, D), jnp.float32),
        pltpu.VMEM_SHARED((SEGP, D), jnp.float32),
        pltpu.SemaphoreType.DMA,
        pltpu.SemaphoreType.DMA,
    ],
)
def _sc_segment_sum(emb_hbm, idx_hbm, out_hbm, idx_v, buf0, buf1, acc, sem0, sem1):
    _sc_body(emb_hbm, idx_hbm, out_hbm, idx_v, buf0, buf1, acc, sem0, sem1)


def _combine_body(a_ref, b_ref, o_ref):
    o_ref[...] = a_ref[0] + b_ref[0]


def kernel(embedding, fragment_regionxcell_ix, cell_n, region_n):
    del cell_n, region_n
    idx3 = fragment_regionxcell_ix.reshape(NW, K, C)
    partials = _sc_segment_sum(embedding, idx3)
    out = pl.pallas_call(
        _combine_body,
        grid=(10,),
        in_specs=[
            pl.BlockSpec((1, SEG // 10, D), lambda i: (0, i, 0)),
            pl.BlockSpec((1, SEG // 10, D), lambda i: (1, i, 0)),
        ],
        out_specs=pl.BlockSpec((SEG // 10, D), lambda i: (i, 0)),
        out_shape=jax.ShapeDtypeStruct((SEG, D), jnp.float32),
    )(partials, partials)
    return out.reshape(10, 1000, D)


# speedup vs baseline: 4.5756x; 1.0516x over previous
import functools

import jax
import jax.numpy as jnp
from jax import lax
from jax.experimental import pallas as pl
from jax.experimental.pallas import tpu as pltpu
from jax.experimental.pallas import tpu_sc as plsc

N = 320000
D = 128
SEG = 10000
SEGP = 10240
NC = 2
NS = 16
NW = NC * NS
ROWS_W = N // NW
C = 128
K = ROWS_W // C
T = ROWS_W - K * C
SEG_T = SEGP // NS


def _sc_body(emb_hbm, idxa_hbm, idxb_hbm, out_hbm, idx_v, idx_t, acc, buf0,
             buf1, sem0, sem1):
    c = lax.axis_index("c")
    s = lax.axis_index("s")
    wid = c * NS + s
    row_base = wid * ROWS_W

    zeros = jnp.zeros((16,), jnp.float32)

    def _zrow(i, _):
        for j in range(D // 16):
            buf0[i, pl.ds(j * 16, 16)] = zeros
        return 0

    lax.fori_loop(0, C, _zrow, 0)
    for r in range(SEG_T // C):
        pltpu.sync_copy(buf0, acc.at[pl.ds(s * SEG_T + r * C, C)])

    pltpu.sync_copy(idxa_hbm.at[wid], idx_v)
    pltpu.sync_copy(idxb_hbm.at[wid], idx_t)

    plsc.subcore_barrier()

    def _gather(g, buf, sem):
        pltpu.async_copy(emb_hbm.at[pl.ds(row_base + g * C, C)], buf, sem)

    def _gwait(buf, sem):
        pltpu.make_async_copy(emb_hbm.at[pl.ds(row_base, C)], buf, sem).wait()

    def _scat(g, buf):
        pltpu.sync_copy(buf, acc.at[idx_v.at[g]], add=True)

    _gather(0, buf0, sem0)

    def _pair(i, _):
        g = 2 * i
        _gather(g + 1, buf1, sem1)
        _gwait(buf0, sem0)
        _scat(g, buf0)
        _gather(g + 2, buf0, sem0)
        _gwait(buf1, sem1)
        _scat(g + 1, buf1)
        return 0

    lax.fori_loop(0, (K - 2) // 2, _pair, 0)
    _gather(K - 1, buf1, sem1)
    _gwait(buf0, sem0)
    _scat(K - 2, buf0)
    pltpu.async_copy(emb_hbm.at[pl.ds(row_base + K * C, T)],
                     buf0.at[pl.ds(0, T)], sem0)
    _gwait(buf1, sem1)
    _scat(K - 1, buf1)
    pltpu.make_async_copy(emb_hbm.at[pl.ds(row_base, T)],
                          buf0.at[pl.ds(0, T)], sem0).wait()
    pltpu.sync_copy(buf0.at[pl.ds(0, T)], acc.at[idx_t.at[0]], add=True)

    plsc.subcore_barrier()

    pltpu.sync_copy(
        acc.at[pl.ds(s * SEG_T, SEG_T)],
        out_hbm.at[c, pl.ds(s * SEG_T, SEG_T)],
    )


@functools.partial(
    pl.kernel,
    mesh=plsc.VectorSubcoreMesh(core_axis_name="c", subcore_axis_name="s"),
    out_type=jax.ShapeDtypeStruct((NC, SEGP, D), jnp.float32),
    scratch_types=[
        pltpu.VMEM((K, C), jnp.int32),
        pltpu.VMEM((1, T), jnp.int32),
        pltpu.VMEM_SHARED((SEGP, D), jnp.float32),
        pltpu.VMEM((C, D), jnp.float32),
        pltpu.VMEM((C, D), jnp.float32),
        pltpu.SemaphoreType.DMA,
        pltpu.SemaphoreType.DMA,
    ],
)
def _sc_segment_sum(emb_hbm, idxa_hbm, idxb_hbm, out_hbm, idx_v, idx_t, acc,
                    buf0, buf1, sem0, sem1):
    _sc_body(emb_hbm, idxa_hbm, idxb_hbm, out_hbm, idx_v, idx_t, acc, buf0,
             buf1, sem0, sem1)


def _combine_body(a_ref, b_ref, o_ref):
    o_ref[...] = a_ref[0] + b_ref[0]


def kernel(embedding, fragment_regionxcell_ix, cell_n, region_n):
    del cell_n, region_n
    idx2 = fragment_regionxcell_ix.reshape(NW, ROWS_W)
    idxa = idx2[:, : K * C].reshape(NW, K, C)
    idxb = idx2[:, K * C :].reshape(NW, 1, T)
    partials = _sc_segment_sum(embedding, idxa, idxb)
    out = pl.pallas_call(
        _combine_body,
        grid=(10,),
        in_specs=[
            pl.BlockSpec((1, SEG // 10, D), lambda i: (0, i, 0)),
            pl.BlockSpec((1, SEG // 10, D), lambda i: (1, i, 0)),
        ],
        out_specs=pl.BlockSpec((SEG // 10, D), lambda i: (i, 0)),
        out_shape=jax.ShapeDtypeStruct((SEG, D), jnp.float32),
    )(partials, partials)
    return out.reshape(10, 1000, D)
